# baseline (speedup 1.0000x reference)
import functools

import jax
import jax.numpy as jnp
from jax import lax
from jax.experimental import pallas as pl
from jax.experimental.pallas import tpu as pltpu
from jax.experimental.pallas import tpu_sc as plsc

NC = 2
NS = 16
NW = NC * NS
LANES = 16
ROWLEN = 144


def _sc_partials(n, e, d, feature, srcf, dstr, wf):
    epw = e // NW
    chunk = 40
    nchunk = epw // chunk
    stage = 50
    nstage = nchunk // stage
    rows_pt = n // NS
    jblocks = d // LANES

    mesh = plsc.VectorSubcoreMesh(core_axis_name="c", subcore_axis_name="s")

    @functools.partial(
        pl.kernel,
        out_type=(jax.ShapeDtypeStruct((NC, n, d), jnp.float32),
                  jax.ShapeDtypeStruct((NC, n, LANES), jnp.float32)),
        mesh=mesh,
        compiler_params=pltpu.CompilerParams(
            use_tc_tiling_on_sc=False, needs_layout_passes=False),
        scratch_types=[
            pltpu.VMEM((stage * chunk,), jnp.int32),
            pltpu.VMEM((stage, chunk), jnp.int32),
            pltpu.VMEM((stage * chunk,), jnp.float32),
            pltpu.VMEM((4, chunk, d), jnp.float32),
            pltpu.VMEM((chunk, LANES), jnp.float32),
            pltpu.VMEM((chunk, LANES), jnp.float32),
            pltpu.VMEM_SHARED((n, d), jnp.float32),
            pltpu.VMEM_SHARED((n, LANES), jnp.float32),
            pltpu.SemaphoreType.DMA,
            pltpu.SemaphoreType.DMA,
            pltpu.SemaphoreType.DMA,
            pltpu.SemaphoreType.DMA,
            pltpu.SemaphoreType.DMA,
            pltpu.SemaphoreType.DMA,
            pltpu.SemaphoreType.DMA,
            pltpu.SemaphoreType.DMA,
            pltpu.SemaphoreType.DMA,
            pltpu.SemaphoreType.DMA,
        ],
    )
    def sc_kernel(feat_hbm, src_hbm, dst_hbm, w_hbm,
                  out_s_hbm, out_d_hbm,
                  src_v, dst_v, w_v, rows_v, ones_v, zero_v, acc_sh, deg_sh,
                  semg0, semg1, semg2, semg3, sems0, sems1, sems2, sems3,
                  semdeg, semidx):
        cid = lax.axis_index("c")
        sid = lax.axis_index("s")
        wid = sid * NC + cid
        semg = (semg0, semg1, semg2, semg3)
        semsc = (sems0, sems1, sems2, sems3)

        ones16 = jnp.where(lax.iota(jnp.int32, LANES) == 0,
                           jnp.float32(1.0), jnp.float32(0.0))
        zero16 = jnp.zeros((LANES,), jnp.float32)

        def init_consts(k, carry):
            ones_v[k, pl.ds(0, LANES)] = ones16
            zero_v[k, pl.ds(0, LANES)] = zero16
            for j in range(jblocks):
                rows_v[0, k, pl.ds(j * LANES, LANES)] = zero16
            return carry
        lax.fori_loop(0, chunk, init_consts, None)

        r0 = sid * rows_pt
        nslab = rows_pt // chunk
        rem = rows_pt - nslab * chunk

        def zero_slab(i, carry):
            pltpu.sync_copy(rows_v.at[0],
                            acc_sh.at[pl.ds(r0 + i * chunk, chunk), :])
            pltpu.sync_copy(zero_v,
                            deg_sh.at[pl.ds(r0 + i * chunk, chunk), :])
            return carry
        lax.fori_loop(0, nslab, zero_slab, None)
        if rem:
            pltpu.sync_copy(rows_v.at[0, pl.ds(0, rem), :],
                            acc_sh.at[pl.ds(r0 + nslab * chunk, rem), :])
            pltpu.sync_copy(zero_v.at[pl.ds(0, rem), :],
                            deg_sh.at[pl.ds(r0 + nslab * chunk, rem), :])
        plsc.subcore_barrier()

        def issue_gather(ci, b):
            pltpu.async_copy(feat_hbm.at[src_v.at[pl.ds(ci * chunk, chunk)]],
                             rows_v.at[b], semg[b])

        def wait_gather(ci, b):
            pltpu.make_async_copy(
                feat_hbm.at[src_v.at[pl.ds(ci * chunk, chunk)]],
                rows_v.at[b], semg[b]).wait()

        def issue_scatter(ci, b):
            pltpu.async_copy(ones_v, deg_sh.at[dst_v.at[ci]], semdeg, add=True)
            pltpu.async_copy(rows_v.at[b], acc_sh.at[dst_v.at[ci]],
                             semsc[b], add=True)

        def wait_scatter(ci, b):
            pltpu.make_async_copy(rows_v.at[b], acc_sh.at[dst_v.at[ci]],
                                  semsc[b]).wait()

        def compute(ci, b):
            def edge_body(k, carry):
                wb = plsc.load_gather(
                    w_v, [jnp.full((LANES,), ci * chunk + k, jnp.int32)])
                for j in range(jblocks):
                    rows_v[b, k, pl.ds(j * LANES, LANES)] = (
                        rows_v[b, k, pl.ds(j * LANES, LANES)] * wb)
                return carry
            lax.fori_loop(0, chunk, edge_body, None, unroll=4)

        def step(ci, b, first=False):
            wait_gather(ci, b)
            compute(ci, b)
            issue_scatter(ci, b)
            if not first:
                wait_scatter(ci - 2, (b + 2) % 4)

            @pl.when(ci + 2 < stage)
            def _():
                issue_gather(ci + 2, (b + 2) % 4)

        def stage_body(s, carry):
            sb = s * stage
            eb = wid * epw + sb * chunk
            pltpu.async_copy(src_hbm.at[pl.ds(eb, stage * chunk)], src_v,
                             semidx)
            pltpu.async_copy(dst_hbm.at[wid, pl.ds(sb, stage), :], dst_v,
                             semidx)
            pltpu.async_copy(w_hbm.at[pl.ds(eb, stage * chunk)], w_v, semidx)
            pltpu.make_async_copy(src_hbm.at[pl.ds(eb, stage * chunk)],
                                  src_v, semidx).wait()
            pltpu.make_async_copy(dst_hbm.at[wid, pl.ds(sb, stage), :],
                                  dst_v, semidx).wait()
            pltpu.make_async_copy(w_hbm.at[pl.ds(eb, stage * chunk)],
                                  w_v, semidx).wait()

            issue_gather(0, 0)
            issue_gather(1, 1)
            step(0, 0, first=True)
            step(1, 1, first=True)

            def quad_body(t, c2):
                ci0 = 4 * t + 2
                step(ci0, 2)
                step(ci0 + 1, 3)
                step(ci0 + 2, 0)
                step(ci0 + 3, 1)
                return c2
            lax.fori_loop(0, (stage - 2) // 4, quad_body, None)

            wait_scatter(stage - 2, (stage - 2) % 4)
            wait_scatter(stage - 1, (stage - 1) % 4)

            def deg_drain(ci, c2):
                pltpu.make_async_copy(ones_v, deg_sh.at[dst_v.at[0]],
                                      semdeg).wait()
                return c2
            lax.fori_loop(0, stage, deg_drain, None)
            return carry
        lax.fori_loop(0, nstage, stage_body, None)

        plsc.subcore_barrier()
        pltpu.sync_copy(acc_sh.at[pl.ds(r0, rows_pt), :],
                        out_s_hbm.at[cid, pl.ds(r0, rows_pt), :])
        pltpu.sync_copy(deg_sh.at[pl.ds(r0, rows_pt), :],
                        out_d_hbm.at[cid, pl.ds(r0, rows_pt), :])

    return sc_kernel(feature, srcf, dstr, wf)


def _tc_combine(n, d, psum, pdeg, feature, w1t, w2t, b2d):
    blk = 1000

    def body(ps_ref, pd_ref, f_ref, w1_ref, w2_ref, b_ref, o_ref):
        s = ps_ref[0] + ps_ref[1]
        deg = pd_ref[0, :, 0:1] + pd_ref[1, :, 0:1]
        r = s / jnp.maximum(deg, 1.0)
        acc = jnp.dot(f_ref[...], w1_ref[...],
                      preferred_element_type=jnp.float32)
        acc = acc + jnp.dot(r, w2_ref[...],
                            preferred_element_type=jnp.float32)
        o_ref[...] = jnp.maximum(acc + b_ref[...], 0.0)

    return pl.pallas_call(
        body,
        grid=(n // blk,),
        in_specs=[
            pl.BlockSpec((NC, blk, d), lambda i: (0, i, 0)),
            pl.BlockSpec((NC, blk, LANES), lambda i: (0, i, 0)),
            pl.BlockSpec((blk, d), lambda i: (i, 0)),
            pl.BlockSpec((d, d), lambda i: (0, 0)),
            pl.BlockSpec((d, d), lambda i: (0, 0)),
            pl.BlockSpec((1, d), lambda i: (0, 0)),
        ],
        out_specs=pl.BlockSpec((blk, d), lambda i: (i, 0)),
        out_shape=jax.ShapeDtypeStruct((n, d), jnp.float32),
    )(psum, pdeg, feature, w1t, w2t, b2d)


@jax.jit
def kernel(feature, edge_index, edge_weight, W, b):
    n, d = feature.shape
    e = edge_index.shape[1]
    src = edge_index[0]
    dst = edge_index[1]

    c = feature[src[0]]
    w1t = W[:, :d].T
    w2t = (W[:, d:] * c[None, :]).T
    b2d = b.reshape(1, d)

    epw = e // NW
    chunk = 40
    dstr = dst.reshape(NW, epw // chunk, chunk)

    psum, pdeg = _sc_partials(n, e, d, feature, src, dstr, edge_weight)
    return _tc_combine(n, d, psum, pdeg, feature, w1t, w2t, b2d)
